# baseline (speedup 1.0000x reference)
import jax
import jax.numpy as jnp
from jax import lax
from jax.experimental import pallas as pl
from jax.experimental.pallas import tpu as pltpu
from jax.experimental.pallas import tpu_sc as plsc

VOCAB = 1000000
SEQ_LEN = 200
EMBED = 32
PAD = 128
LROW = 128
BATCH = 4096

NC, NS = 2, 16
NW = NC * NS
B_PER_W = BATCH // NW
CB = 8
NPANELS = BATCH // CB
NCHUNKS = B_PER_W // CB
SPLIT = 128
REST = SEQ_LEN - SPLIT

_MESH = plsc.VectorSubcoreMesh(
    core_axis_name="c", subcore_axis_name="s", num_cores=NC, num_subcores=NS
)


def _body(tok_hbm, idxa_hbm, idxb_hbm, pos_hbm, out_hbm, idxa_v, idxb_v,
          rows_v, pos_v, sem):
    wid = lax.axis_index("s") * NC + lax.axis_index("c")
    base_b = wid * B_PER_W

    pltpu.sync_copy(pos_hbm, pos_v)

    def chunk_body(ci, _):
        b0 = pl.multiple_of(base_b + ci * CB, CB)
        pltpu.sync_copy(idxa_hbm.at[pl.ds(b0, CB)], idxa_v)
        pltpu.sync_copy(idxb_hbm.at[pl.ds(b0, CB)], idxb_v)
        for r in range(CB):
            pltpu.async_copy(
                tok_hbm.at[idxa_v.at[r]],
                rows_v.at[r, pl.ds(0, SPLIT)],
                sem,
            )
            pltpu.async_copy(
                tok_hbm.at[idxb_v.at[r, pl.ds(0, REST)]],
                rows_v.at[r, pl.ds(SPLIT, REST)],
                sem,
            )
        for r in range(CB):
            pltpu.make_async_copy(
                tok_hbm.at[idxa_v.at[r]],
                rows_v.at[r, pl.ds(0, SPLIT)],
                sem,
            ).wait()
            pltpu.make_async_copy(
                tok_hbm.at[idxb_v.at[r, pl.ds(0, REST)]],
                rows_v.at[r, pl.ds(SPLIT, REST)],
                sem,
            ).wait()

        def add_body(l, _):
            p0 = pos_v[l, 0:16]
            p1 = pos_v[l, 16:32]
            for b in range(CB):
                rows_v[b, l, 0:16] = rows_v[b, l, 0:16] + p0
                rows_v[b, l, 16:32] = rows_v[b, l, 16:32] + p1
            return 0

        lax.fori_loop(0, SEQ_LEN, add_body, 0)

        pltpu.sync_copy(
            rows_v, out_hbm.at[pl.ds(b0, CB), slice(None), pl.ds(0, EMBED)]
        )
        return 0

    lax.fori_loop(0, NCHUNKS, chunk_body, 0)


@jax.jit
def _run(tok, idx, pos):
    idxp = jnp.pad(idx, ((0, 0), (0, 2 * SPLIT - SEQ_LEN)))
    idxa = idxp[:, :SPLIT]
    idxb = idxp[:, SPLIT:]
    out = pl.kernel(
        _body,
        out_type=jax.ShapeDtypeStruct((BATCH, SEQ_LEN, PAD), jnp.float32),
        mesh=_MESH,
        scratch_types=[
            pltpu.VMEM((CB, LROW), jnp.int32),
            pltpu.VMEM((CB, LROW), jnp.int32),
            pltpu.VMEM((CB, SEQ_LEN, EMBED), jnp.float32),
            pltpu.VMEM((SEQ_LEN, EMBED), jnp.float32),
            pltpu.SemaphoreType.DMA,
        ],
        compiler_params=pltpu.CompilerParams(use_tc_tiling_on_sc=False),
    )(tok, idxa, idxb, pos)
    return out[..., :EMBED]


def kernel(inputs, token_table, pos_table):
    return _run(token_table, inputs, pos_table)


# speedup vs baseline: 1.0003x; 1.0003x over previous
import jax
import jax.numpy as jnp
from jax import lax
from jax.experimental import pallas as pl
from jax.experimental.pallas import tpu as pltpu
from jax.experimental.pallas import tpu_sc as plsc

VOCAB = 1000000
SEQ_LEN = 200
EMBED = 32
PAD = 128
LROW = 128
BATCH = 4096

NC, NS = 2, 16
NW = NC * NS
B_PER_W = BATCH // NW
CB = 8
NPANELS = BATCH // CB
NCHUNKS = B_PER_W // CB
SPLIT = 128
REST = SEQ_LEN - SPLIT

_MESH = plsc.VectorSubcoreMesh(
    core_axis_name="c", subcore_axis_name="s", num_cores=NC, num_subcores=NS
)


def _body(tok_hbm, idxa_hbm, idxb_hbm, pos_hbm, out_hbm, idxa_v, idxb_v,
          rows_v, pos_v, sem):
    wid = lax.axis_index("s") * NC + lax.axis_index("c")
    base_b = wid * B_PER_W

    pltpu.sync_copy(pos_hbm, pos_v)

    def chunk_body(ci, _):
        b0 = pl.multiple_of(base_b + ci * CB, CB)
        pltpu.sync_copy(idxa_hbm.at[pl.ds(b0, CB)], idxa_v)
        pltpu.sync_copy(idxb_hbm.at[pl.ds(b0, CB)], idxb_v)
        for r in range(CB):
            pltpu.async_copy(
                tok_hbm.at[idxa_v.at[r]],
                rows_v.at[r, pl.ds(0, SPLIT)],
                sem,
            )
            pltpu.async_copy(
                tok_hbm.at[idxb_v.at[r, pl.ds(0, REST)]],
                rows_v.at[r, pl.ds(SPLIT, REST)],
                sem,
            )
        for r in range(CB):
            pltpu.make_async_copy(
                tok_hbm.at[idxa_v.at[r]],
                rows_v.at[r, pl.ds(0, SPLIT)],
                sem,
            ).wait()
            pltpu.make_async_copy(
                tok_hbm.at[idxb_v.at[r, pl.ds(0, REST)]],
                rows_v.at[r, pl.ds(SPLIT, REST)],
                sem,
            ).wait()

        def add_body(l, _):
            p0 = pos_v[l, 0:16]
            p1 = pos_v[l, 16:32]
            for b in range(CB):
                rows_v[b, l, 0:16] = rows_v[b, l, 0:16] + p0
                rows_v[b, l, 16:32] = rows_v[b, l, 16:32] + p1
            return 0

        lax.fori_loop(0, SEQ_LEN, add_body, 0)

        pltpu.sync_copy(
            rows_v, out_hbm.at[pl.ds(b0, CB), slice(None), pl.ds(0, EMBED)]
        )
        return 0

    lax.fori_loop(0, NCHUNKS, chunk_body, 0)


@jax.jit
def _run(tok, idx, pos):
    tok = lax.optimization_barrier(tok.reshape(-1)).reshape(VOCAB, EMBED)
    idxp = jnp.pad(idx, ((0, 0), (0, 2 * SPLIT - SEQ_LEN)))
    idxa = idxp[:, :SPLIT]
    idxb = idxp[:, SPLIT:]
    out = pl.kernel(
        _body,
        out_type=jax.ShapeDtypeStruct((BATCH, SEQ_LEN, PAD), jnp.float32),
        mesh=_MESH,
        scratch_types=[
            pltpu.VMEM((CB, LROW), jnp.int32),
            pltpu.VMEM((CB, LROW), jnp.int32),
            pltpu.VMEM((CB, SEQ_LEN, EMBED), jnp.float32),
            pltpu.VMEM((SEQ_LEN, EMBED), jnp.float32),
            pltpu.SemaphoreType.DMA,
        ],
        compiler_params=pltpu.CompilerParams(use_tc_tiling_on_sc=False),
    )(tok, idxa, idxb, pos)
    return out[..., :EMBED]


def kernel(inputs, token_table, pos_table):
    return _run(token_table, inputs, pos_table)


# speedup vs baseline: 1.0550x; 1.0547x over previous
import jax
import jax.numpy as jnp
from jax import lax
from jax.experimental import pallas as pl
from jax.experimental.pallas import tpu as pltpu
from jax.experimental.pallas import tpu_sc as plsc

VOCAB = 1000000
SEQ_LEN = 200
EMBED = 32
PAD = 128
LROW = 128
BATCH = 4096

NC, NS = 2, 16
NW = NC * NS
B_PER_W = BATCH // NW
CB = 8
NCHUNKS = B_PER_W // CB
SPLIT = 128
REST = SEQ_LEN - SPLIT
NBUF = 2

_MESH = plsc.VectorSubcoreMesh(
    core_axis_name="c", subcore_axis_name="s", num_cores=NC, num_subcores=NS
)


def _body(tok_hbm, idxa_hbm, idxb_hbm, pos_hbm, out_hbm,
          idxa_v, idxb_v, rows_v, pos_v, sems):
    wid = lax.axis_index("s") * NC + lax.axis_index("c")
    base_b = wid * B_PER_W

    pltpu.sync_copy(pos_hbm, pos_v)

    def fetch(ci, par):
        b0 = pl.multiple_of(base_b + ci * CB, CB)
        pltpu.sync_copy(idxa_hbm.at[pl.ds(b0, CB)], idxa_v.at[par])
        pltpu.sync_copy(idxb_hbm.at[pl.ds(b0, CB)], idxb_v.at[par])
        for r in range(CB):
            pltpu.async_copy(
                tok_hbm.at[idxa_v.at[par, r]],
                rows_v.at[par, r, pl.ds(0, SPLIT)],
                sems.at[par],
            )
            pltpu.async_copy(
                tok_hbm.at[idxb_v.at[par, r, pl.ds(0, REST)]],
                rows_v.at[par, r, pl.ds(SPLIT, REST)],
                sems.at[par],
            )

    def drain(ci, par):
        for r in range(CB):
            pltpu.make_async_copy(
                tok_hbm.at[idxa_v.at[par, r]],
                rows_v.at[par, r, pl.ds(0, SPLIT)],
                sems.at[par],
            ).wait()
            pltpu.make_async_copy(
                tok_hbm.at[idxb_v.at[par, r, pl.ds(0, REST)]],
                rows_v.at[par, r, pl.ds(SPLIT, REST)],
                sems.at[par],
            ).wait()

    for par in range(NBUF):
        fetch(par, par)

    def ring_body(j, _):
        for par in range(NBUF):
            ci = NBUF * j + par
            b0 = pl.multiple_of(base_b + ci * CB, CB)
            drain(ci, par)

            def add_body(l, _):
                p0 = pos_v[l, 0:16]
                p1 = pos_v[l, 16:32]
                for b in range(CB):
                    rows_v[par, b, l, 0:16] = rows_v[par, b, l, 0:16] + p0
                    rows_v[par, b, l, 16:32] = rows_v[par, b, l, 16:32] + p1
                return 0

            lax.fori_loop(0, SEQ_LEN, add_body, 0)

            pltpu.sync_copy(
                rows_v.at[par],
                out_hbm.at[pl.ds(b0, CB), slice(None), pl.ds(0, EMBED)],
            )

            @pl.when(ci + NBUF < NCHUNKS)
            def _():
                fetch(ci + NBUF, par)

        return 0

    lax.fori_loop(0, NCHUNKS // NBUF, ring_body, 0)


@jax.jit
def _run(tok, idx, pos):
    idxp = jnp.pad(idx, ((0, 0), (0, 2 * SPLIT - SEQ_LEN)))
    idxa = idxp[:, :SPLIT]
    idxb = idxp[:, SPLIT:]
    out = pl.kernel(
        _body,
        out_type=jax.ShapeDtypeStruct((BATCH, SEQ_LEN, PAD), jnp.float32),
        mesh=_MESH,
        scratch_types=[
            pltpu.VMEM((NBUF, CB, LROW), jnp.int32),
            pltpu.VMEM((NBUF, CB, LROW), jnp.int32),
            pltpu.VMEM((NBUF, CB, SEQ_LEN, EMBED), jnp.float32),
            pltpu.VMEM((SEQ_LEN, EMBED), jnp.float32),
            pltpu.SemaphoreType.DMA((NBUF,)),
        ],
        compiler_params=pltpu.CompilerParams(use_tc_tiling_on_sc=False),
    )(tok, idxa, idxb, pos)
    return out[..., :EMBED]


def kernel(inputs, token_table, pos_table):
    return _run(token_table, inputs, pos_table)
